# baseline (speedup 1.0000x reference)
import functools

import jax
import jax.numpy as jnp
from jax import lax
from jax.experimental import pallas as pl
from jax.experimental.pallas import tpu as pltpu
from jax.experimental.pallas import tpu_sc as plsc

B, L, K, D = 1024, 50, 26, 64
GRID = 16
BT = B // GRID
_INV_K = 1.0 / K


def _sc_emb_fn(emb_hbm, tidx_hbm, e_hbm, idx_v, rows_v, e_v, sem):
    wid = lax.axis_index("s") + lax.axis_index("c")

    @pl.when(wid == 0)
    def _():
        pltpu.sync_copy(tidx_hbm, idx_v)
        pltpu.async_copy(emb_hbm.at[idx_v], rows_v, sem).wait()
        for m in range(D // 16):
            sl = pl.ds(m * 16, 16)
            acc = rows_v[0, sl]
            for k in range(1, K):
                acc = acc + rows_v[k, sl]
            e_v[sl] = acc
        pltpu.sync_copy(e_v, e_hbm)


def _sc_emb_sum(emb_pad, tidx_flat):
    mesh = plsc.VectorSubcoreMesh(core_axis_name="c", subcore_axis_name="s",
                                  num_cores=1)
    f = functools.partial(
        pl.kernel, mesh=mesh,
        out_type=jax.ShapeDtypeStruct((D,), jnp.float32),
        scratch_types=[
            pltpu.VMEM((K,), jnp.int32),
            pltpu.VMEM((K, 128), jnp.float32),
            pltpu.VMEM((D,), jnp.float32),
            pltpu.SemaphoreType.DMA,
        ],
    )(_sc_emb_fn)
    return f(emb_pad, tidx_flat)


def _tc_reduce_body(ev_ref, s_ref):
    s_ref[...] = jnp.sum(ev_ref[...], axis=2)


def _tc_expand_body(s_ref, e_ref, wv_ref, bv_ref, out_ref):
    s = s_ref[...][:, :, None]
    wv = (wv_ref[...] * _INV_K).reshape(1, 1, D)
    c = (bv_ref[...] + e_ref[...] * _INV_K).reshape(1, 1, D)
    out_ref[...] = s * wv + c


def kernel(event_time, event_value, non_pad_mask, w_val, b_val, emb_table,
           w_per, b_per, w_lin, b_lin, k_map, type_idx):
    s = pl.pallas_call(
        _tc_reduce_body,
        grid=(GRID,),
        in_specs=[pl.BlockSpec((BT, L, K), lambda i: (i, 0, 0))],
        out_specs=pl.BlockSpec((BT, L), lambda i: (i, 0)),
        out_shape=jax.ShapeDtypeStruct((B, L), jnp.float32),
    )(event_value)
    emb_pad = jnp.pad(emb_table, ((0, 0), (0, 128 - D)))
    e = _sc_emb_sum(emb_pad, type_idx.reshape(K))
    return pl.pallas_call(
        _tc_expand_body,
        grid=(GRID,),
        in_specs=[
            pl.BlockSpec((BT, L), lambda i: (i, 0)),
            pl.BlockSpec((D,), lambda i: (0,)),
            pl.BlockSpec((D,), lambda i: (0,)),
            pl.BlockSpec((D,), lambda i: (0,)),
        ],
        out_specs=pl.BlockSpec((BT, L, D), lambda i: (i, 0, 0)),
        out_shape=jax.ShapeDtypeStruct((B, L, D), jnp.float32),
    )(s, e, w_val, b_val)


# speedup vs baseline: 1.1758x; 1.1758x over previous
import functools

import jax
import jax.numpy as jnp
from jax import lax
from jax.experimental import pallas as pl
from jax.experimental.pallas import tpu as pltpu
from jax.experimental.pallas import tpu_sc as plsc

B, L, K, D = 1024, 50, 26, 64
GRID = 8
BT = B // GRID
_INV_K = 1.0 / K


def _sc_emb_fn(emb_hbm, tidx_hbm, e_hbm, idx_v, rows_v, e_v, sem):
    wid = lax.axis_index("s") + lax.axis_index("c")

    @pl.when(wid == 0)
    def _():
        pltpu.sync_copy(tidx_hbm.at[0, 0], idx_v)
        pltpu.async_copy(emb_hbm.at[idx_v], rows_v, sem).wait()
        for m in range(D // 16):
            sl = pl.ds(m * 16, 16)
            acc = rows_v[0, sl]
            for k in range(1, K):
                acc = acc + rows_v[k, sl]
            e_v[sl] = acc
        pltpu.sync_copy(e_v, e_hbm)


def _sc_emb_sum(emb_pad, tidx_flat):
    mesh = plsc.VectorSubcoreMesh(core_axis_name="c", subcore_axis_name="s",
                                  num_cores=1)
    f = functools.partial(
        pl.kernel, mesh=mesh,
        out_type=jax.ShapeDtypeStruct((D,), jnp.float32),
        scratch_types=[
            pltpu.VMEM((K,), jnp.int32),
            pltpu.VMEM((K, 128), jnp.float32),
            pltpu.VMEM((D,), jnp.float32),
            pltpu.SemaphoreType.DMA,
        ],
    )(_sc_emb_fn)
    return f(emb_pad, tidx_flat)


def _tc_body(ev_ref, e_ref, wv_ref, bv_ref, out_ref):
    s = jnp.sum(ev_ref[...], axis=2, keepdims=True)
    wv = (wv_ref[...] * _INV_K).reshape(1, 1, D)
    c = (bv_ref[...] + e_ref[...] * _INV_K).reshape(1, 1, D)
    out_ref[...] = s * wv + c


def kernel(event_time, event_value, non_pad_mask, w_val, b_val, emb_table,
           w_per, b_per, w_lin, b_lin, k_map, type_idx):
    emb_pad = jnp.pad(emb_table, ((0, 0), (0, 128 - D)))
    e = _sc_emb_sum(emb_pad, type_idx)
    return pl.pallas_call(
        _tc_body,
        grid=(GRID,),
        in_specs=[
            pl.BlockSpec((BT, L, K), lambda i: (i, 0, 0)),
            pl.BlockSpec((D,), lambda i: (0,)),
            pl.BlockSpec((D,), lambda i: (0,)),
            pl.BlockSpec((D,), lambda i: (0,)),
        ],
        out_specs=pl.BlockSpec((BT, L, D), lambda i: (i, 0, 0)),
        out_shape=jax.ShapeDtypeStruct((B, L, D), jnp.float32),
    )(event_value, e, w_val, b_val)


# speedup vs baseline: 1.1839x; 1.0069x over previous
import functools

import jax
import jax.numpy as jnp
from jax import lax
from jax.experimental import pallas as pl
from jax.experimental.pallas import tpu as pltpu
from jax.experimental.pallas import tpu_sc as plsc

B, L, K, D = 1024, 50, 26, 64
GRID = 4
BT = B // GRID
_INV_K = 1.0 / K


def _sc_emb_fn(emb_hbm, tidx_hbm, e_hbm, idx_v, rows_v, e_v, sem):
    wid = lax.axis_index("s") + lax.axis_index("c")

    @pl.when(wid == 0)
    def _():
        pltpu.sync_copy(tidx_hbm.at[0, 0], idx_v)
        pltpu.async_copy(emb_hbm.at[idx_v], rows_v, sem).wait()
        for m in range(D // 16):
            sl = pl.ds(m * 16, 16)
            acc = rows_v[0, sl]
            for k in range(1, K):
                acc = acc + rows_v[k, sl]
            e_v[sl] = acc
        pltpu.sync_copy(e_v, e_hbm)


def _sc_emb_sum(emb_pad, tidx_flat):
    mesh = plsc.VectorSubcoreMesh(core_axis_name="c", subcore_axis_name="s",
                                  num_cores=1)
    f = functools.partial(
        pl.kernel, mesh=mesh,
        out_type=jax.ShapeDtypeStruct((D,), jnp.float32),
        scratch_types=[
            pltpu.VMEM((K,), jnp.int32),
            pltpu.VMEM((K, 128), jnp.float32),
            pltpu.VMEM((D,), jnp.float32),
            pltpu.SemaphoreType.DMA,
        ],
    )(_sc_emb_fn)
    return f(emb_pad, tidx_flat)


def _tc_body(ev_ref, e_ref, wv_ref, bv_ref, out_ref):
    s = jnp.sum(ev_ref[...], axis=2, keepdims=True)
    wv = (wv_ref[...] * _INV_K).reshape(1, 1, D)
    c = (bv_ref[...] + e_ref[...] * _INV_K).reshape(1, 1, D)
    out_ref[...] = s * wv + c


def kernel(event_time, event_value, non_pad_mask, w_val, b_val, emb_table,
           w_per, b_per, w_lin, b_lin, k_map, type_idx):
    emb_pad = jnp.pad(emb_table, ((0, 0), (0, 128 - D)))
    e = _sc_emb_sum(emb_pad, type_idx)
    return pl.pallas_call(
        _tc_body,
        grid=(GRID,),
        in_specs=[
            pl.BlockSpec((BT, L, K), lambda i: (i, 0, 0)),
            pl.BlockSpec((D,), lambda i: (0,)),
            pl.BlockSpec((D,), lambda i: (0,)),
            pl.BlockSpec((D,), lambda i: (0,)),
        ],
        out_specs=pl.BlockSpec((BT, L, D), lambda i: (i, 0, 0)),
        out_shape=jax.ShapeDtypeStruct((B, L, D), jnp.float32),
    )(event_value, e, w_val, b_val)


# speedup vs baseline: 1.2247x; 1.0345x over previous
import functools

import jax
import jax.numpy as jnp
from jax import lax
from jax.experimental import pallas as pl
from jax.experimental.pallas import tpu as pltpu
from jax.experimental.pallas import tpu_sc as plsc

B, L, K, D = 1024, 50, 26, 64
GRID = 2
BT = B // GRID
_INV_K = 1.0 / K


def _sc_emb_fn(emb_hbm, tidx_hbm, e_hbm, idx_v, rows_v, e_v, sem):
    wid = lax.axis_index("s") + lax.axis_index("c")

    @pl.when(wid == 0)
    def _():
        pltpu.sync_copy(tidx_hbm.at[0, 0], idx_v)
        pltpu.async_copy(emb_hbm.at[idx_v], rows_v, sem).wait()
        for m in range(D // 16):
            sl = pl.ds(m * 16, 16)
            acc = rows_v[0, sl]
            for k in range(1, K):
                acc = acc + rows_v[k, sl]
            e_v[sl] = acc
        pltpu.sync_copy(e_v, e_hbm)


def _sc_emb_sum(emb_pad, tidx_flat):
    mesh = plsc.VectorSubcoreMesh(core_axis_name="c", subcore_axis_name="s",
                                  num_cores=1)
    f = functools.partial(
        pl.kernel, mesh=mesh,
        out_type=jax.ShapeDtypeStruct((D,), jnp.float32),
        scratch_types=[
            pltpu.VMEM((K,), jnp.int32),
            pltpu.VMEM((K, 128), jnp.float32),
            pltpu.VMEM((D,), jnp.float32),
            pltpu.SemaphoreType.DMA,
        ],
    )(_sc_emb_fn)
    return f(emb_pad, tidx_flat)


def _tc_body(ev_ref, e_ref, wv_ref, bv_ref, out_ref):
    s = jnp.sum(ev_ref[...], axis=2, keepdims=True)
    wv = (wv_ref[...] * _INV_K).reshape(1, 1, D)
    c = (bv_ref[...] + e_ref[...] * _INV_K).reshape(1, 1, D)
    out_ref[...] = s * wv + c


def kernel(event_time, event_value, non_pad_mask, w_val, b_val, emb_table,
           w_per, b_per, w_lin, b_lin, k_map, type_idx):
    emb_pad = jnp.pad(emb_table, ((0, 0), (0, 128 - D)))
    e = _sc_emb_sum(emb_pad, type_idx)
    return pl.pallas_call(
        _tc_body,
        grid=(GRID,),
        in_specs=[
            pl.BlockSpec((BT, L, K), lambda i: (i, 0, 0)),
            pl.BlockSpec((D,), lambda i: (0,)),
            pl.BlockSpec((D,), lambda i: (0,)),
            pl.BlockSpec((D,), lambda i: (0,)),
        ],
        out_specs=pl.BlockSpec((BT, L, D), lambda i: (i, 0, 0)),
        out_shape=jax.ShapeDtypeStruct((B, L, D), jnp.float32),
    )(event_value, e, w_val, b_val)
